# Initial kernel scaffold; baseline (speedup 1.0000x reference)
import jax
import jax.numpy as jnp
from jax.experimental import pallas as pl


def kernel(input, target):
    raise NotImplementedError("write your pallas kernel here")


# speedup vs baseline: 25.9729x; 25.9729x over previous
import functools

import jax
import jax.numpy as jnp
from jax import lax
from jax.experimental import pallas as pl
from jax.experimental.pallas import tpu as pltpu
from jax.experimental.pallas import tpu_sc as plsc

_N = 32 * 512 * 512
_QIDX = int((1.0 - 0.2) * _N)
_K = _N - _QIDX

_L = 16
_NW = 32
_CHUNK = _N // _NW
_BLK = 16384
_NBLK = _CHUNK // _BLK

_NB1 = 1024
_SH1 = 22
_NB2 = 2048
_SH2 = 11

def _mesh():
    return plsc.VectorSubcoreMesh(
        core_axis_name="c", subcore_axis_name="s", num_cores=2, num_subcores=16)


def _mse_body(inp_ref, tgt_ref, out_ref):
    d = (inp_ref[0] - tgt_ref[0]) * 255.0
    out_ref[0] = (d[0] * d[0] + d[1] * d[1] + d[2] * d[2]) * (1.0 / 3.0)


def _mse(inp, tgt):
    return pl.pallas_call(
        _mse_body,
        grid=(32,),
        in_specs=[
            pl.BlockSpec((1, 3, 512, 512), lambda i: (i, 0, 0, 0)),
            pl.BlockSpec((1, 3, 512, 512), lambda i: (i, 0, 0, 0)),
        ],
        out_specs=pl.BlockSpec((1, 512, 512), lambda i: (i, 0, 0)),
        out_shape=jax.ShapeDtypeStruct((32, 512, 512), jnp.float32),
    )(inp, tgt)


def _wid():
    return lax.axis_index("s") * 2 + lax.axis_index("c")


def _zero_hist(cnt_h, sum_h, nb):
    zi = jnp.zeros((_L,), jnp.int32)
    zf = jnp.zeros((_L,), jnp.float32)

    @pl.loop(0, nb)
    def _z(i):
        cnt_h[pl.ds(i * _L, _L)] = zi
        sum_h[pl.ds(i * _L, _L)] = zf


def _merge_lanes(cnt_h, sum_h, mc, ms, nb):
    @pl.loop(0, nb // _L)
    def _m(g):
        accc = cnt_h[pl.ds(g * _L, _L)]
        accs = sum_h[pl.ds(g * _L, _L)]
        for l in range(1, _L):
            accc = accc + cnt_h[pl.ds(l * nb + g * _L, _L)]
            accs = accs + sum_h[pl.ds(l * nb + g * _L, _L)]
        mc[pl.ds(g * _L, _L)] = accc
        ms[pl.ds(g * _L, _L)] = accs


def _hist1_body(e_hbm, cnt_out, sum_out, buf, cnt_h, sum_h, mc, ms):
    wid = _wid()
    lane_off = lax.iota(jnp.int32, _L) * _NB1
    ones = jnp.ones((_L,), jnp.int32)
    sh1 = jnp.full((_L,), _SH1, jnp.int32)

    _zero_hist(cnt_h, sum_h, _NB1)

    base = wid * _CHUNK
    for blk in range(_NBLK):
        pltpu.sync_copy(e_hbm.at[pl.ds(base + blk * _BLK, _BLK)], buf)

        @pl.loop(0, _BLK // _L)
        def _acc(j):
            v = buf[pl.ds(j * _L, _L)]
            u = plsc.bitcast(v, jnp.int32)
            idx = lax.shift_right_logical(u, sh1) + lane_off
            plsc.addupdate_scatter(cnt_h, [idx], ones)
            plsc.addupdate_scatter(sum_h, [idx], v)

    _merge_lanes(cnt_h, sum_h, mc, ms, _NB1)
    pltpu.sync_copy(mc, cnt_out.at[wid])
    pltpu.sync_copy(ms, sum_out.at[wid])


def _hist2_body(e_hbm, b_hbm, cnt_out, sum_out, buf, bv_v, cnt_h, sum_h, mc, ms):
    wid = _wid()
    lane_off = lax.iota(jnp.int32, _L) * _NB2
    ones = jnp.ones((_L,), jnp.int32)
    sh1 = jnp.full((_L,), _SH1, jnp.int32)
    sh2 = jnp.full((_L,), _SH2, jnp.int32)
    msk2 = jnp.full((_L,), _NB2 - 1, jnp.int32)

    pltpu.sync_copy(b_hbm, bv_v)
    bstar = bv_v[...]

    _zero_hist(cnt_h, sum_h, _NB2)

    base = wid * _CHUNK
    for blk in range(_NBLK):
        pltpu.sync_copy(e_hbm.at[pl.ds(base + blk * _BLK, _BLK)], buf)

        @pl.loop(0, _BLK // _L)
        def _acc(j):
            v = buf[pl.ds(j * _L, _L)]
            u = plsc.bitcast(v, jnp.int32)
            hit = lax.shift_right_logical(u, sh1) == bstar
            idx = (lax.shift_right_logical(u, sh2) & msk2) + lane_off
            plsc.addupdate_scatter(cnt_h, [idx], ones, mask=hit)
            plsc.addupdate_scatter(sum_h, [idx], v, mask=hit)

    _merge_lanes(cnt_h, sum_h, mc, ms, _NB2)
    pltpu.sync_copy(mc, cnt_out.at[wid])
    pltpu.sync_copy(ms, sum_out.at[wid])


@functools.cache
def _hist1():
    return pl.kernel(
        _hist1_body,
        out_type=(jax.ShapeDtypeStruct((_NW, _NB1), jnp.int32),
                  jax.ShapeDtypeStruct((_NW, _NB1), jnp.float32)),
        mesh=_mesh(),
        compiler_params=pltpu.CompilerParams(needs_layout_passes=False),
        scratch_types=[
            pltpu.VMEM((_BLK,), jnp.float32),
            pltpu.VMEM((_L * _NB1,), jnp.int32),
            pltpu.VMEM((_L * _NB1,), jnp.float32),
            pltpu.VMEM((_NB1,), jnp.int32),
            pltpu.VMEM((_NB1,), jnp.float32),
        ],
    )


@functools.cache
def _hist2():
    return pl.kernel(
        _hist2_body,
        out_type=(jax.ShapeDtypeStruct((_NW, _NB2), jnp.int32),
                  jax.ShapeDtypeStruct((_NW, _NB2), jnp.float32)),
        mesh=_mesh(),
        compiler_params=pltpu.CompilerParams(needs_layout_passes=False),
        scratch_types=[
            pltpu.VMEM((_BLK,), jnp.float32),
            pltpu.VMEM((_L,), jnp.int32),
            pltpu.VMEM((_L * _NB2,), jnp.int32),
            pltpu.VMEM((_L * _NB2,), jnp.float32),
            pltpu.VMEM((_NB2,), jnp.int32),
            pltpu.VMEM((_NB2,), jnp.float32),
        ],
    )


def _suffix(x):
    s = jnp.cumsum(x[::-1])[::-1].astype(x.dtype)
    return jnp.concatenate([s, jnp.zeros((1,), x.dtype)])


def kernel(input, target):
    e = _mse(input, target).reshape(_N)

    c1, s1 = _hist1()(e)
    cnt1 = c1.sum(axis=0)
    sum1 = s1.sum(axis=0)
    sfx_c1 = _suffix(cnt1)
    sfx_s1 = _suffix(sum1)
    bstar = jnp.sum((sfx_c1[:_NB1] >= _K).astype(jnp.int32)) - 1
    c_above1 = sfx_c1[bstar + 1]
    s_above1 = sfx_s1[bstar + 1]
    m = _K - c_above1

    c2, s2 = _hist2()(e, jnp.full((_L,), bstar, jnp.int32))
    cnt2 = c2.sum(axis=0)
    sum2 = s2.sum(axis=0)
    sfx_c2 = _suffix(cnt2)
    sfx_s2 = _suffix(sum2)
    b2 = jnp.sum((sfx_c2[:_NB2] >= m).astype(jnp.int32)) - 1
    c_above2 = sfx_c2[b2 + 1]
    s_above2 = sfx_s2[b2 + 1]
    m2 = (m - c_above2).astype(jnp.float32)
    mean_b2 = sum2[b2] / jnp.maximum(cnt2[b2].astype(jnp.float32), 1.0)

    total = s_above1 + s_above2 + m2 * mean_b2
    return (total / jnp.float32(_K)).astype(jnp.float32)


# speedup vs baseline: 27.7833x; 1.0697x over previous
import functools

import jax
import jax.numpy as jnp
from jax import lax
from jax.experimental import pallas as pl
from jax.experimental.pallas import tpu as pltpu
from jax.experimental.pallas import tpu_sc as plsc

_N = 32 * 512 * 512
_QIDX = int((1.0 - 0.2) * _N)
_K = _N - _QIDX

_L = 16
_NW = 32
_CHUNK = _N // _NW
_BLK = 16384
_NBLK = _CHUNK // _BLK

_NB1 = 1024
_SH1 = 22
_NB2 = 2048
_SH2 = 11

def _mesh():
    return plsc.VectorSubcoreMesh(
        core_axis_name="c", subcore_axis_name="s", num_cores=2, num_subcores=16)


def _mse_body(inp_ref, tgt_ref, out_ref):
    d = (inp_ref[0] - tgt_ref[0]) * 255.0
    out_ref[0] = (d[0] * d[0] + d[1] * d[1] + d[2] * d[2]) * (1.0 / 3.0)


def _mse(inp, tgt):
    return pl.pallas_call(
        _mse_body,
        grid=(32,),
        in_specs=[
            pl.BlockSpec((1, 3, 512, 512), lambda i: (i, 0, 0, 0)),
            pl.BlockSpec((1, 3, 512, 512), lambda i: (i, 0, 0, 0)),
        ],
        out_specs=pl.BlockSpec((1, 512, 512), lambda i: (i, 0, 0)),
        out_shape=jax.ShapeDtypeStruct((32, 512, 512), jnp.float32),
    )(inp, tgt)


def _wid():
    return lax.axis_index("s") * 2 + lax.axis_index("c")


def _zero_hist(cnt_h, sum_h, nb):
    zi = jnp.zeros((_L,), jnp.int32)
    zf = jnp.zeros((_L,), jnp.float32)

    @pl.loop(0, nb)
    def _z(i):
        cnt_h[pl.ds(i * _L, _L)] = zi
        sum_h[pl.ds(i * _L, _L)] = zf


def _merge_lanes(cnt_h, sum_h, mc, ms, nb):
    @pl.loop(0, nb // _L)
    def _m(g):
        accc = cnt_h[pl.ds(g * _L, _L)]
        accs = sum_h[pl.ds(g * _L, _L)]
        for l in range(1, _L):
            accc = accc + cnt_h[pl.ds(l * nb + g * _L, _L)]
            accs = accs + sum_h[pl.ds(l * nb + g * _L, _L)]
        mc[pl.ds(g * _L, _L)] = accc
        ms[pl.ds(g * _L, _L)] = accs


def _hist1_body(e_hbm, cnt_out, sum_out, buf0, buf1, sem0, sem1,
                cnt_h, sum_h, mc, ms):
    wid = _wid()
    lane_off = lax.iota(jnp.int32, _L) * _NB1
    ones = jnp.ones((_L,), jnp.int32)
    sh1 = jnp.full((_L,), _SH1, jnp.int32)

    _zero_hist(cnt_h, sum_h, _NB1)

    base = wid * _CHUNK
    bufs = (buf0, buf1)
    sems = (sem0, sem1)
    cps = [None, None]
    cps[0] = pltpu.async_copy(e_hbm.at[pl.ds(base, _BLK)], buf0, sem0)
    for blk in range(_NBLK):
        cur = blk % 2
        if blk + 1 < _NBLK:
            cps[1 - cur] = pltpu.async_copy(
                e_hbm.at[pl.ds(base + (blk + 1) * _BLK, _BLK)],
                bufs[1 - cur], sems[1 - cur])
        cps[cur].wait()
        buf = bufs[cur]

        @pl.loop(0, _BLK // _L, unroll=8)
        def _acc(j):
            v = buf[pl.ds(j * _L, _L)]
            u = plsc.bitcast(v, jnp.int32)
            idx = lax.shift_right_logical(u, sh1) + lane_off
            plsc.addupdate_scatter(cnt_h, [idx], ones)
            plsc.addupdate_scatter(sum_h, [idx], v)

    _merge_lanes(cnt_h, sum_h, mc, ms, _NB1)
    pltpu.sync_copy(mc, cnt_out.at[wid])
    pltpu.sync_copy(ms, sum_out.at[wid])


def _hist2_body(e_hbm, b_hbm, cnt_out, sum_out, buf0, buf1, sem0, sem1,
                bv_v, cnt_h, sum_h, mc, ms):
    wid = _wid()
    lane_off = lax.iota(jnp.int32, _L) * _NB2
    ones = jnp.ones((_L,), jnp.int32)
    sh1 = jnp.full((_L,), _SH1, jnp.int32)
    sh2 = jnp.full((_L,), _SH2, jnp.int32)
    msk2 = jnp.full((_L,), _NB2 - 1, jnp.int32)

    pltpu.sync_copy(b_hbm, bv_v)
    bstar = bv_v[...]

    _zero_hist(cnt_h, sum_h, _NB2)

    base = wid * _CHUNK
    bufs = (buf0, buf1)
    sems = (sem0, sem1)
    cps = [None, None]
    cps[0] = pltpu.async_copy(e_hbm.at[pl.ds(base, _BLK)], buf0, sem0)
    for blk in range(_NBLK):
        cur = blk % 2
        if blk + 1 < _NBLK:
            cps[1 - cur] = pltpu.async_copy(
                e_hbm.at[pl.ds(base + (blk + 1) * _BLK, _BLK)],
                bufs[1 - cur], sems[1 - cur])
        cps[cur].wait()
        buf = bufs[cur]

        @pl.loop(0, _BLK // _L, unroll=8)
        def _acc(j):
            v = buf[pl.ds(j * _L, _L)]
            u = plsc.bitcast(v, jnp.int32)
            hit = lax.shift_right_logical(u, sh1) == bstar
            idx = (lax.shift_right_logical(u, sh2) & msk2) + lane_off
            plsc.addupdate_scatter(cnt_h, [idx], ones, mask=hit)
            plsc.addupdate_scatter(sum_h, [idx], v, mask=hit)

    _merge_lanes(cnt_h, sum_h, mc, ms, _NB2)
    pltpu.sync_copy(mc, cnt_out.at[wid])
    pltpu.sync_copy(ms, sum_out.at[wid])


@functools.cache
def _hist1():
    return pl.kernel(
        _hist1_body,
        out_type=(jax.ShapeDtypeStruct((_NW, _NB1), jnp.int32),
                  jax.ShapeDtypeStruct((_NW, _NB1), jnp.float32)),
        mesh=_mesh(),
        compiler_params=pltpu.CompilerParams(needs_layout_passes=False),
        scratch_types=[
            pltpu.VMEM((_BLK,), jnp.float32),
            pltpu.VMEM((_BLK,), jnp.float32),
            pltpu.SemaphoreType.DMA,
            pltpu.SemaphoreType.DMA,
            pltpu.VMEM((_L * _NB1,), jnp.int32),
            pltpu.VMEM((_L * _NB1,), jnp.float32),
            pltpu.VMEM((_NB1,), jnp.int32),
            pltpu.VMEM((_NB1,), jnp.float32),
        ],
    )


@functools.cache
def _hist2():
    return pl.kernel(
        _hist2_body,
        out_type=(jax.ShapeDtypeStruct((_NW, _NB2), jnp.int32),
                  jax.ShapeDtypeStruct((_NW, _NB2), jnp.float32)),
        mesh=_mesh(),
        compiler_params=pltpu.CompilerParams(needs_layout_passes=False),
        scratch_types=[
            pltpu.VMEM((_BLK,), jnp.float32),
            pltpu.VMEM((_BLK,), jnp.float32),
            pltpu.SemaphoreType.DMA,
            pltpu.SemaphoreType.DMA,
            pltpu.VMEM((_L,), jnp.int32),
            pltpu.VMEM((_L * _NB2,), jnp.int32),
            pltpu.VMEM((_L * _NB2,), jnp.float32),
            pltpu.VMEM((_NB2,), jnp.int32),
            pltpu.VMEM((_NB2,), jnp.float32),
        ],
    )


def _suffix(x):
    s = jnp.cumsum(x[::-1])[::-1].astype(x.dtype)
    return jnp.concatenate([s, jnp.zeros((1,), x.dtype)])


def kernel(input, target):
    e = _mse(input, target).reshape(_N)

    c1, s1 = _hist1()(e)
    cnt1 = c1.sum(axis=0)
    sum1 = s1.sum(axis=0)
    sfx_c1 = _suffix(cnt1)
    sfx_s1 = _suffix(sum1)
    bstar = jnp.sum((sfx_c1[:_NB1] >= _K).astype(jnp.int32)) - 1
    c_above1 = sfx_c1[bstar + 1]
    s_above1 = sfx_s1[bstar + 1]
    m = _K - c_above1

    c2, s2 = _hist2()(e, jnp.full((_L,), bstar, jnp.int32))
    cnt2 = c2.sum(axis=0)
    sum2 = s2.sum(axis=0)
    sfx_c2 = _suffix(cnt2)
    sfx_s2 = _suffix(sum2)
    b2 = jnp.sum((sfx_c2[:_NB2] >= m).astype(jnp.int32)) - 1
    c_above2 = sfx_c2[b2 + 1]
    s_above2 = sfx_s2[b2 + 1]
    m2 = (m - c_above2).astype(jnp.float32)
    mean_b2 = sum2[b2] / jnp.maximum(cnt2[b2].astype(jnp.float32), 1.0)

    total = s_above1 + s_above2 + m2 * mean_b2
    return (total / jnp.float32(_K)).astype(jnp.float32)


# speedup vs baseline: 46.7604x; 1.6830x over previous
import functools

import jax
import jax.numpy as jnp
from jax import lax
from jax.experimental import pallas as pl
from jax.experimental.pallas import tpu as pltpu
from jax.experimental.pallas import tpu_sc as plsc

_N = 32 * 512 * 512
_QIDX = int((1.0 - 0.2) * _N)
_K = _N - _QIDX

_L = 16
_NW = 32
_CHUNK = _N // _NW
_BLK = 16384
_NBLK = _CHUNK // _BLK

_NB1 = 1024
_SH1 = 22
_NB2 = 2048
_SH2 = 11
_VPB = 8

def _mesh():
    return plsc.VectorSubcoreMesh(
        core_axis_name="c", subcore_axis_name="s", num_cores=2, num_subcores=16)


def _mse_body(inp_ref, tgt_ref, out_ref):
    d = (inp_ref[0] - tgt_ref[0]) * 255.0
    out_ref[0] = (d[0] * d[0] + d[1] * d[1] + d[2] * d[2]) * (1.0 / 3.0)


def _mse(inp, tgt):
    return pl.pallas_call(
        _mse_body,
        grid=(32,),
        in_specs=[
            pl.BlockSpec((1, 3, 512, 512), lambda i: (i, 0, 0, 0)),
            pl.BlockSpec((1, 3, 512, 512), lambda i: (i, 0, 0, 0)),
        ],
        out_specs=pl.BlockSpec((1, 512, 512), lambda i: (i, 0, 0)),
        out_shape=jax.ShapeDtypeStruct((32, 512, 512), jnp.float32),
    )(inp, tgt)


def _wid():
    return lax.axis_index("s") * 2 + lax.axis_index("c")


def _zero_hist(cnt_h, sum_h, nb):
    zi = jnp.zeros((_L,), jnp.int32)
    zf = jnp.zeros((_L,), jnp.float32)

    @pl.loop(0, nb)
    def _z(i):
        cnt_h[pl.ds(i * _L, _L)] = zi
        sum_h[pl.ds(i * _L, _L)] = zf


def _merge_lanes(cnt_h, sum_h, mc, ms, nb):
    @pl.loop(0, nb // _L)
    def _m(g):
        accc = cnt_h[pl.ds(g * _L, _L)]
        accs = sum_h[pl.ds(g * _L, _L)]
        for l in range(1, _L):
            accc = accc + cnt_h[pl.ds(l * nb + g * _L, _L)]
            accs = accs + sum_h[pl.ds(l * nb + g * _L, _L)]
        mc[pl.ds(g * _L, _L)] = accc
        ms[pl.ds(g * _L, _L)] = accs


def _hist1_body(e_hbm, cnt_out, sum_out, buf0, buf1, sem0, sem1,
                cnt_h, sum_h, mc, ms):
    wid = _wid()
    lane_off = lax.iota(jnp.int32, _L) * _NB1
    ones = jnp.ones((_L,), jnp.int32)
    sh1 = jnp.full((_L,), _SH1, jnp.int32)

    _zero_hist(cnt_h, sum_h, _NB1)

    base = wid * _CHUNK
    bufs = (buf0, buf1)
    sems = (sem0, sem1)
    cps = [None, None]
    cps[0] = pltpu.async_copy(e_hbm.at[pl.ds(base, _BLK)], buf0, sem0)
    for blk in range(_NBLK):
        cur = blk % 2
        if blk + 1 < _NBLK:
            cps[1 - cur] = pltpu.async_copy(
                e_hbm.at[pl.ds(base + (blk + 1) * _BLK, _BLK)],
                bufs[1 - cur], sems[1 - cur])
        cps[cur].wait()
        buf = bufs[cur]

        @pl.loop(0, _BLK // (_L * _VPB))
        def _acc(g):
            j0 = g * (_L * _VPB)
            vs = [buf[pl.ds(j0 + t * _L, _L)] for t in range(_VPB)]
            idxs = [lax.shift_right_logical(plsc.bitcast(v, jnp.int32), sh1)
                    + lane_off for v in vs]
            for t in range(_VPB):
                plsc.addupdate_scatter(cnt_h, [idxs[t]], ones)
                plsc.addupdate_scatter(sum_h, [idxs[t]], vs[t])

    _merge_lanes(cnt_h, sum_h, mc, ms, _NB1)
    pltpu.sync_copy(mc, cnt_out.at[wid])
    pltpu.sync_copy(ms, sum_out.at[wid])


def _hist2_body(e_hbm, b_hbm, cnt_out, sum_out, buf0, buf1, sem0, sem1,
                bv_v, cnt_h, sum_h, mc, ms):
    wid = _wid()
    lane_off = lax.iota(jnp.int32, _L) * _NB2
    ones = jnp.ones((_L,), jnp.int32)
    sh1 = jnp.full((_L,), _SH1, jnp.int32)
    sh2 = jnp.full((_L,), _SH2, jnp.int32)
    msk2 = jnp.full((_L,), _NB2 - 1, jnp.int32)

    pltpu.sync_copy(b_hbm, bv_v)
    bstar = bv_v[...]

    _zero_hist(cnt_h, sum_h, _NB2)

    base = wid * _CHUNK
    bufs = (buf0, buf1)
    sems = (sem0, sem1)
    cps = [None, None]
    cps[0] = pltpu.async_copy(e_hbm.at[pl.ds(base, _BLK)], buf0, sem0)
    for blk in range(_NBLK):
        cur = blk % 2
        if blk + 1 < _NBLK:
            cps[1 - cur] = pltpu.async_copy(
                e_hbm.at[pl.ds(base + (blk + 1) * _BLK, _BLK)],
                bufs[1 - cur], sems[1 - cur])
        cps[cur].wait()
        buf = bufs[cur]

        @pl.loop(0, _BLK // (_L * _VPB))
        def _acc(g):
            j0 = g * (_L * _VPB)
            vs = [buf[pl.ds(j0 + t * _L, _L)] for t in range(_VPB)]
            us = [plsc.bitcast(v, jnp.int32) for v in vs]
            hits = [lax.shift_right_logical(u, sh1) == bstar for u in us]
            idxs = [(lax.shift_right_logical(u, sh2) & msk2) + lane_off
                    for u in us]
            for t in range(_VPB):
                plsc.addupdate_scatter(cnt_h, [idxs[t]], ones, mask=hits[t])
                plsc.addupdate_scatter(sum_h, [idxs[t]], vs[t], mask=hits[t])

    _merge_lanes(cnt_h, sum_h, mc, ms, _NB2)
    pltpu.sync_copy(mc, cnt_out.at[wid])
    pltpu.sync_copy(ms, sum_out.at[wid])


@functools.cache
def _hist1():
    return pl.kernel(
        _hist1_body,
        out_type=(jax.ShapeDtypeStruct((_NW, _NB1), jnp.int32),
                  jax.ShapeDtypeStruct((_NW, _NB1), jnp.float32)),
        mesh=_mesh(),
        compiler_params=pltpu.CompilerParams(needs_layout_passes=False),
        scratch_types=[
            pltpu.VMEM((_BLK,), jnp.float32),
            pltpu.VMEM((_BLK,), jnp.float32),
            pltpu.SemaphoreType.DMA,
            pltpu.SemaphoreType.DMA,
            pltpu.VMEM((_L * _NB1,), jnp.int32),
            pltpu.VMEM((_L * _NB1,), jnp.float32),
            pltpu.VMEM((_NB1,), jnp.int32),
            pltpu.VMEM((_NB1,), jnp.float32),
        ],
    )


@functools.cache
def _hist2():
    return pl.kernel(
        _hist2_body,
        out_type=(jax.ShapeDtypeStruct((_NW, _NB2), jnp.int32),
                  jax.ShapeDtypeStruct((_NW, _NB2), jnp.float32)),
        mesh=_mesh(),
        compiler_params=pltpu.CompilerParams(needs_layout_passes=False),
        scratch_types=[
            pltpu.VMEM((_BLK,), jnp.float32),
            pltpu.VMEM((_BLK,), jnp.float32),
            pltpu.SemaphoreType.DMA,
            pltpu.SemaphoreType.DMA,
            pltpu.VMEM((_L,), jnp.int32),
            pltpu.VMEM((_L * _NB2,), jnp.int32),
            pltpu.VMEM((_L * _NB2,), jnp.float32),
            pltpu.VMEM((_NB2,), jnp.int32),
            pltpu.VMEM((_NB2,), jnp.float32),
        ],
    )


def _suffix(x):
    s = jnp.cumsum(x[::-1])[::-1].astype(x.dtype)
    return jnp.concatenate([s, jnp.zeros((1,), x.dtype)])


def kernel(input, target):
    e = _mse(input, target).reshape(_N)

    c1, s1 = _hist1()(e)
    cnt1 = c1.sum(axis=0)
    sum1 = s1.sum(axis=0)
    sfx_c1 = _suffix(cnt1)
    sfx_s1 = _suffix(sum1)
    bstar = jnp.sum((sfx_c1[:_NB1] >= _K).astype(jnp.int32)) - 1
    c_above1 = sfx_c1[bstar + 1]
    s_above1 = sfx_s1[bstar + 1]
    m = _K - c_above1

    c2, s2 = _hist2()(e, jnp.full((_L,), bstar, jnp.int32))
    cnt2 = c2.sum(axis=0)
    sum2 = s2.sum(axis=0)
    sfx_c2 = _suffix(cnt2)
    sfx_s2 = _suffix(sum2)
    b2 = jnp.sum((sfx_c2[:_NB2] >= m).astype(jnp.int32)) - 1
    c_above2 = sfx_c2[b2 + 1]
    s_above2 = sfx_s2[b2 + 1]
    m2 = (m - c_above2).astype(jnp.float32)
    mean_b2 = sum2[b2] / jnp.maximum(cnt2[b2].astype(jnp.float32), 1.0)

    total = s_above1 + s_above2 + m2 * mean_b2
    return (total / jnp.float32(_K)).astype(jnp.float32)


# speedup vs baseline: 53.4292x; 1.1426x over previous
import functools

import jax
import jax.numpy as jnp
from jax import lax
from jax.experimental import pallas as pl
from jax.experimental.pallas import tpu as pltpu
from jax.experimental.pallas import tpu_sc as plsc

_N = 32 * 512 * 512
_QIDX = int((1.0 - 0.2) * _N)
_K = _N - _QIDX

_L = 16
_NW = 32
_CHUNK = _N // _NW
_BLK = 16384
_NBLK = _CHUNK // _BLK

_NB1 = 1024
_SH1 = 22
_NB2 = 2048
_SH2 = 11
_VPB = 8

def _mesh():
    return plsc.VectorSubcoreMesh(
        core_axis_name="c", subcore_axis_name="s", num_cores=2, num_subcores=16)


def _mse_body(inp_ref, tgt_ref, out_ref):
    d = (inp_ref[0] - tgt_ref[0]) * 255.0
    out_ref[0] = (d[0] * d[0] + d[1] * d[1] + d[2] * d[2]) * (1.0 / 3.0)


def _mse(inp, tgt):
    return pl.pallas_call(
        _mse_body,
        grid=(32,),
        in_specs=[
            pl.BlockSpec((1, 3, 512, 512), lambda i: (i, 0, 0, 0)),
            pl.BlockSpec((1, 3, 512, 512), lambda i: (i, 0, 0, 0)),
        ],
        out_specs=pl.BlockSpec((1, 512, 512), lambda i: (i, 0, 0)),
        out_shape=jax.ShapeDtypeStruct((32, 512, 512), jnp.float32),
    )(inp, tgt)


def _wid():
    return lax.axis_index("s") * 2 + lax.axis_index("c")


def _zero_hist(cnt_h, sum_h, nb):
    zi = jnp.zeros((_L,), jnp.int32)
    zf = jnp.zeros((_L,), jnp.float32)

    @pl.loop(0, (_L * (nb + 1) + _L - 1) // _L)
    def _z(i):
        cnt_h[pl.ds(i * _L, _L)] = zi
        sum_h[pl.ds(i * _L, _L)] = zf


def _merge_lanes(cnt_h, sum_h, mc, ms, nb):
    @pl.loop(0, nb // _L)
    def _m(g):
        accc = cnt_h[pl.ds(g * _L, _L)]
        accs = sum_h[pl.ds(g * _L, _L)]
        for l in range(1, _L):
            accc = accc + cnt_h[pl.ds(l * (nb + 1) + g * _L, _L)]
            accs = accs + sum_h[pl.ds(l * (nb + 1) + g * _L, _L)]
        mc[pl.ds(g * _L, _L)] = accc
        ms[pl.ds(g * _L, _L)] = accs


def _hist1_body(e_hbm, cnt_out, sum_out, buf0, buf1, sem0, sem1,
                cnt_h, sum_h, mc, ms):
    wid = _wid()
    lane_off = lax.iota(jnp.int32, _L) * (_NB1 + 1)
    ones = jnp.ones((_L,), jnp.int32)
    sh1 = jnp.full((_L,), _SH1, jnp.int32)

    _zero_hist(cnt_h, sum_h, _NB1)

    base = wid * _CHUNK
    bufs = (buf0, buf1)
    sems = (sem0, sem1)
    cps = [None, None]
    cps[0] = pltpu.async_copy(e_hbm.at[pl.ds(base, _BLK)], buf0, sem0)
    for blk in range(_NBLK):
        cur = blk % 2
        if blk + 1 < _NBLK:
            cps[1 - cur] = pltpu.async_copy(
                e_hbm.at[pl.ds(base + (blk + 1) * _BLK, _BLK)],
                bufs[1 - cur], sems[1 - cur])
        cps[cur].wait()
        buf = bufs[cur]

        @pl.loop(0, _BLK // (_L * _VPB))
        def _acc(g):
            j0 = g * (_L * _VPB)
            vs = [buf[pl.ds(j0 + t * _L, _L)] for t in range(_VPB)]
            idxs = [lax.shift_right_logical(plsc.bitcast(v, jnp.int32), sh1)
                    + lane_off for v in vs]
            for t in range(_VPB):
                plsc.addupdate_scatter(cnt_h, [idxs[t]], ones)
                plsc.addupdate_scatter(sum_h, [idxs[t]], vs[t])

    _merge_lanes(cnt_h, sum_h, mc, ms, _NB1)
    pltpu.sync_copy(mc, cnt_out.at[wid])
    pltpu.sync_copy(ms, sum_out.at[wid])


def _hist2_body(e_hbm, b_hbm, cnt_out, sum_out, buf0, buf1, sem0, sem1,
                bv_v, cnt_h, sum_h, mc, ms):
    wid = _wid()
    lane_off = lax.iota(jnp.int32, _L) * (_NB2 + 1)
    ones = jnp.ones((_L,), jnp.int32)
    sh1 = jnp.full((_L,), _SH1, jnp.int32)
    sh2 = jnp.full((_L,), _SH2, jnp.int32)
    msk2 = jnp.full((_L,), _NB2 - 1, jnp.int32)

    pltpu.sync_copy(b_hbm, bv_v)
    bstar = bv_v[...]

    _zero_hist(cnt_h, sum_h, _NB2)

    base = wid * _CHUNK
    bufs = (buf0, buf1)
    sems = (sem0, sem1)
    cps = [None, None]
    cps[0] = pltpu.async_copy(e_hbm.at[pl.ds(base, _BLK)], buf0, sem0)
    for blk in range(_NBLK):
        cur = blk % 2
        if blk + 1 < _NBLK:
            cps[1 - cur] = pltpu.async_copy(
                e_hbm.at[pl.ds(base + (blk + 1) * _BLK, _BLK)],
                bufs[1 - cur], sems[1 - cur])
        cps[cur].wait()
        buf = bufs[cur]

        @pl.loop(0, _BLK // (_L * _VPB))
        def _acc(g):
            j0 = g * (_L * _VPB)
            vs = [buf[pl.ds(j0 + t * _L, _L)] for t in range(_VPB)]
            us = [plsc.bitcast(v, jnp.int32) for v in vs]
            hits = [lax.shift_right_logical(u, sh1) == bstar for u in us]
            idxs = [(lax.shift_right_logical(u, sh2) & msk2) + lane_off
                    for u in us]
            for t in range(_VPB):
                plsc.addupdate_scatter(cnt_h, [idxs[t]], ones, mask=hits[t])
                plsc.addupdate_scatter(sum_h, [idxs[t]], vs[t], mask=hits[t])

    _merge_lanes(cnt_h, sum_h, mc, ms, _NB2)
    pltpu.sync_copy(mc, cnt_out.at[wid])
    pltpu.sync_copy(ms, sum_out.at[wid])


@functools.cache
def _hist1():
    return pl.kernel(
        _hist1_body,
        out_type=(jax.ShapeDtypeStruct((_NW, _NB1), jnp.int32),
                  jax.ShapeDtypeStruct((_NW, _NB1), jnp.float32)),
        mesh=_mesh(),
        compiler_params=pltpu.CompilerParams(needs_layout_passes=False),
        scratch_types=[
            pltpu.VMEM((_BLK,), jnp.float32),
            pltpu.VMEM((_BLK,), jnp.float32),
            pltpu.SemaphoreType.DMA,
            pltpu.SemaphoreType.DMA,
            pltpu.VMEM((_L * (_NB1 + 1),), jnp.int32),
            pltpu.VMEM((_L * (_NB1 + 1),), jnp.float32),
            pltpu.VMEM((_NB1,), jnp.int32),
            pltpu.VMEM((_NB1,), jnp.float32),
        ],
    )


@functools.cache
def _hist2():
    return pl.kernel(
        _hist2_body,
        out_type=(jax.ShapeDtypeStruct((_NW, _NB2), jnp.int32),
                  jax.ShapeDtypeStruct((_NW, _NB2), jnp.float32)),
        mesh=_mesh(),
        compiler_params=pltpu.CompilerParams(needs_layout_passes=False),
        scratch_types=[
            pltpu.VMEM((_BLK,), jnp.float32),
            pltpu.VMEM((_BLK,), jnp.float32),
            pltpu.SemaphoreType.DMA,
            pltpu.SemaphoreType.DMA,
            pltpu.VMEM((_L,), jnp.int32),
            pltpu.VMEM((_L * (_NB2 + 1),), jnp.int32),
            pltpu.VMEM((_L * (_NB2 + 1),), jnp.float32),
            pltpu.VMEM((_NB2,), jnp.int32),
            pltpu.VMEM((_NB2,), jnp.float32),
        ],
    )


def _suffix(x):
    s = jnp.cumsum(x[::-1])[::-1].astype(x.dtype)
    return jnp.concatenate([s, jnp.zeros((1,), x.dtype)])


def kernel(input, target):
    e = _mse(input, target).reshape(_N)

    c1, s1 = _hist1()(e)
    cnt1 = c1.sum(axis=0)
    sum1 = s1.sum(axis=0)
    sfx_c1 = _suffix(cnt1)
    sfx_s1 = _suffix(sum1)
    bstar = jnp.sum((sfx_c1[:_NB1] >= _K).astype(jnp.int32)) - 1
    c_above1 = sfx_c1[bstar + 1]
    s_above1 = sfx_s1[bstar + 1]
    m = _K - c_above1

    c2, s2 = _hist2()(e, jnp.full((_L,), bstar, jnp.int32))
    cnt2 = c2.sum(axis=0)
    sum2 = s2.sum(axis=0)
    sfx_c2 = _suffix(cnt2)
    sfx_s2 = _suffix(sum2)
    b2 = jnp.sum((sfx_c2[:_NB2] >= m).astype(jnp.int32)) - 1
    c_above2 = sfx_c2[b2 + 1]
    s_above2 = sfx_s2[b2 + 1]
    m2 = (m - c_above2).astype(jnp.float32)
    mean_b2 = sum2[b2] / jnp.maximum(cnt2[b2].astype(jnp.float32), 1.0)

    total = s_above1 + s_above2 + m2 * mean_b2
    return (total / jnp.float32(_K)).astype(jnp.float32)


# speedup vs baseline: 60.2298x; 1.1273x over previous
import functools

import jax
import jax.numpy as jnp
from jax import lax
from jax.experimental import pallas as pl
from jax.experimental.pallas import tpu as pltpu
from jax.experimental.pallas import tpu_sc as plsc

_N = 32 * 512 * 512
_QIDX = int((1.0 - 0.2) * _N)
_K = _N - _QIDX

_L = 16
_NW = 32
_CHUNK = _N // _NW
_BLK = 16384
_NBLK = _CHUNK // _BLK

_NB1 = 1024
_SH1 = 22
_NB2 = 2048
_SH2 = 11
_VPB = 16


def _mesh():
    return plsc.VectorSubcoreMesh(
        core_axis_name="c", subcore_axis_name="s", num_cores=2, num_subcores=16)


def _mse_body(inp_ref, tgt_ref, out_ref):
    d = (inp_ref[0] - tgt_ref[0]) * 255.0
    out_ref[0] = (d[0] * d[0] + d[1] * d[1] + d[2] * d[2]) * (1.0 / 3.0)


def _mse(inp, tgt):
    return pl.pallas_call(
        _mse_body,
        grid=(32,),
        in_specs=[
            pl.BlockSpec((1, 3, 512, 512), lambda i: (i, 0, 0, 0)),
            pl.BlockSpec((1, 3, 512, 512), lambda i: (i, 0, 0, 0)),
        ],
        out_specs=pl.BlockSpec((1, 512, 512), lambda i: (i, 0, 0)),
        out_shape=jax.ShapeDtypeStruct((32, 512, 512), jnp.float32),
    )(inp, tgt)


def _wid():
    return lax.axis_index("s") * 2 + lax.axis_index("c")


def _zero(refs, nwords):
    @pl.loop(0, nwords // _L)
    def _z(i):
        for r in refs:
            r[pl.ds(i * _L, _L)] = jnp.zeros((_L,), r.dtype)


def _merge_lanes(hists, outs, nb):
    @pl.loop(0, nb // _L)
    def _m(g):
        for h, o in zip(hists, outs):
            acc = h[pl.ds(g * _L, _L)]
            for l in range(1, _L):
                acc = acc + h[pl.ds(l * (nb + 1) + g * _L, _L)]
            o[pl.ds(g * _L, _L)] = acc


def _stage(e_hbm, base, bufs, sems):
    cps = [None, None]
    cps[0] = pltpu.async_copy(e_hbm.at[pl.ds(base, _BLK)], bufs[0], sems[0])

    def gen():
        for blk in range(_NBLK):
            cur = blk % 2
            if blk + 1 < _NBLK:
                cps[1 - cur] = pltpu.async_copy(
                    e_hbm.at[pl.ds(base + (blk + 1) * _BLK, _BLK)],
                    bufs[1 - cur], sems[1 - cur])
            cps[cur].wait()
            yield bufs[cur]

    return gen()


def _hist1_body(e_hbm, cnt_out, buf0, buf1, sem0, sem1, cnt_h, mc):
    wid = _wid()
    lane_off = lax.iota(jnp.int32, _L) * (_NB1 + 1)
    ones = jnp.ones((_L,), jnp.int32)
    sh1 = jnp.full((_L,), _SH1, jnp.int32)

    _zero([cnt_h], _L * (_NB1 + 1))

    for buf in _stage(e_hbm, wid * _CHUNK, (buf0, buf1), (sem0, sem1)):
        @pl.loop(0, _BLK // (_L * _VPB))
        def _acc(g):
            j0 = g * (_L * _VPB)
            vs = [buf[pl.ds(j0 + t * _L, _L)] for t in range(_VPB)]
            idxs = [lax.shift_right_logical(plsc.bitcast(v, jnp.int32), sh1)
                    + lane_off for v in vs]
            for t in range(_VPB):
                plsc.addupdate_scatter(cnt_h, [idxs[t]], ones)

    _merge_lanes([cnt_h], [mc], _NB1)
    pltpu.sync_copy(mc, cnt_out.at[wid])


def _hist2_body(e_hbm, b_hbm, cnt_out, sum_out, acc_out,
                buf0, buf1, sem0, sem1, bv_v, cnt_h, sum_h, mc, ms, macc):
    wid = _wid()
    lane_off = lax.iota(jnp.int32, _L) * (_NB2 + 1)
    ones = jnp.ones((_L,), jnp.int32)
    sh1 = jnp.full((_L,), _SH1, jnp.int32)
    sh2 = jnp.full((_L,), _SH2, jnp.int32)
    msk2 = jnp.full((_L,), _NB2 - 1, jnp.int32)
    zf = jnp.zeros((_L,), jnp.float32)

    pltpu.sync_copy(b_hbm, bv_v)
    bstar = bv_v[...]

    _zero([cnt_h, sum_h], _L * (_NB2 + 1))

    sacc = zf
    for buf in _stage(e_hbm, wid * _CHUNK, (buf0, buf1), (sem0, sem1)):
        @pl.loop(0, _BLK // (_L * _VPB), init_carry=sacc)
        def _acc(g, carry):
            j0 = g * (_L * _VPB)
            vs = [buf[pl.ds(j0 + t * _L, _L)] for t in range(_VPB)]
            us = [plsc.bitcast(v, jnp.int32) for v in vs]
            b1s = [lax.shift_right_logical(u, sh1) for u in us]
            hits = [b1 == bstar for b1 in b1s]
            idxs = [(lax.shift_right_logical(u, sh2) & msk2) + lane_off
                    for u in us]
            for t in range(_VPB):
                plsc.addupdate_scatter(cnt_h, [idxs[t]], ones, mask=hits[t])
                plsc.addupdate_scatter(sum_h, [idxs[t]], vs[t], mask=hits[t])
            for t in range(_VPB):
                carry = carry + jnp.where(b1s[t] > bstar, vs[t], zf)
            return carry

        sacc = _acc

    _merge_lanes([cnt_h, sum_h], [mc, ms], _NB2)
    macc[pl.ds(0, _L)] = sacc
    pltpu.sync_copy(mc, cnt_out.at[wid])
    pltpu.sync_copy(ms, sum_out.at[wid])
    pltpu.sync_copy(macc, acc_out.at[wid])


@functools.cache
def _hist1():
    return pl.kernel(
        _hist1_body,
        out_type=jax.ShapeDtypeStruct((_NW, _NB1), jnp.int32),
        mesh=_mesh(),
        compiler_params=pltpu.CompilerParams(needs_layout_passes=False),
        scratch_types=[
            pltpu.VMEM((_BLK,), jnp.float32),
            pltpu.VMEM((_BLK,), jnp.float32),
            pltpu.SemaphoreType.DMA,
            pltpu.SemaphoreType.DMA,
            pltpu.VMEM((_L * (_NB1 + 1),), jnp.int32),
            pltpu.VMEM((_NB1,), jnp.int32),
        ],
    )


@functools.cache
def _hist2():
    return pl.kernel(
        _hist2_body,
        out_type=(jax.ShapeDtypeStruct((_NW, _NB2), jnp.int32),
                  jax.ShapeDtypeStruct((_NW, _NB2), jnp.float32),
                  jax.ShapeDtypeStruct((_NW, _L), jnp.float32)),
        mesh=_mesh(),
        compiler_params=pltpu.CompilerParams(needs_layout_passes=False),
        scratch_types=[
            pltpu.VMEM((_BLK,), jnp.float32),
            pltpu.VMEM((_BLK,), jnp.float32),
            pltpu.SemaphoreType.DMA,
            pltpu.SemaphoreType.DMA,
            pltpu.VMEM((_L,), jnp.int32),
            pltpu.VMEM((_L * (_NB2 + 1),), jnp.int32),
            pltpu.VMEM((_L * (_NB2 + 1),), jnp.float32),
            pltpu.VMEM((_NB2,), jnp.int32),
            pltpu.VMEM((_NB2,), jnp.float32),
            pltpu.VMEM((_L,), jnp.float32),
        ],
    )


def _suffix(x):
    s = jnp.cumsum(x[::-1])[::-1].astype(x.dtype)
    return jnp.concatenate([s, jnp.zeros((1,), x.dtype)])


def kernel(input, target):
    e = _mse(input, target).reshape(_N)

    c1 = _hist1()(e)
    cnt1 = c1.sum(axis=0)
    sfx_c1 = _suffix(cnt1)
    bstar = jnp.sum((sfx_c1[:_NB1] >= _K).astype(jnp.int32)) - 1
    c_above1 = sfx_c1[bstar + 1]
    m = _K - c_above1

    c2, s2, a2 = _hist2()(e, jnp.full((_L,), bstar, jnp.int32))
    s_above1 = a2.sum()
    cnt2 = c2.sum(axis=0)
    sum2 = s2.sum(axis=0)
    sfx_c2 = _suffix(cnt2)
    sfx_s2 = _suffix(sum2)
    b2 = jnp.sum((sfx_c2[:_NB2] >= m).astype(jnp.int32)) - 1
    c_above2 = sfx_c2[b2 + 1]
    s_above2 = sfx_s2[b2 + 1]
    m2 = (m - c_above2).astype(jnp.float32)
    mean_b2 = sum2[b2] / jnp.maximum(cnt2[b2].astype(jnp.float32), 1.0)

    total = s_above1 + s_above2 + m2 * mean_b2
    return (total / jnp.float32(_K)).astype(jnp.float32)
